# baseline (speedup 1.0000x reference)
import functools

import jax
import jax.numpy as jnp
from jax.experimental import pallas as pl
from jax.experimental.pallas import tpu as pltpu
from jax.experimental.pallas import tpu_sc as plsc

N = 16384
D = 256
K = 8192
BLK = 512
R = N // BLK

_EPS = 1e-12
_LOSS_SCALE = 1.25 / (N * D)


def _l2n(a):
    n = jnp.linalg.norm(a, axis=-1, keepdims=True)
    return a / jnp.maximum(n, _EPS)


def _argmin_body(nsteps, xn_ref, emb_ref, t1_ref, c_ref, idx_ref, loss_ref,
                 ebf_ref):
    r = pl.program_id(0)

    @pl.when(r == 0)
    def _():
        ebf_ref[...] = emb_ref[...].astype(jnp.bfloat16)
        loss_ref[0] = 0.0

    xn = xn_ref[...]
    dots = jax.lax.dot_general(
        xn.astype(jnp.bfloat16), ebf_ref[...],
        (((1,), (1,)), ((), ())),
        preferred_element_type=jnp.float32)
    d = t1_ref[...] + c_ref[...] - 2.0 * dots

    b1 = K // 2
    m0 = jnp.min(d[:, :b1], axis=1)
    i0 = jnp.argmin(d[:, :b1], axis=1).astype(jnp.int32)
    m1 = jnp.min(d[:, b1:], axis=1)
    i1 = jnp.argmin(d[:, b1:], axis=1).astype(jnp.int32) + b1

    def _round_bf16(v):
        u = jax.lax.bitcast_convert_type(v, jnp.uint32)
        lsb = jax.lax.shift_right_logical(u, jnp.uint32(16)) & jnp.uint32(1)
        rr = (u + jnp.uint32(0x7FFF) + lsb) & jnp.uint32(0xFFFF0000)
        return jax.lax.bitcast_convert_type(rr, jnp.float32)

    upd = m1 < _round_bf16(m0)
    idx_ref[0, 0, :] = jnp.where(upd, i1, i0)
    loss_ref[0] += jnp.sum(jnp.where(upd, m1, m0))

    @pl.when(r == nsteps - 1)
    def _():
        loss_ref[0] = loss_ref[0] * _LOSS_SCALE


def _argmin_call(xn, ewn, t1, c):
    rows = xn.shape[0]
    return pl.pallas_call(
        functools.partial(_argmin_body, rows // BLK),
        grid=(rows // BLK,),
        in_specs=[
            pl.BlockSpec((BLK, D), lambda r: (r, 0)),
            pl.BlockSpec((K, D), lambda r: (0, 0)),
            pl.BlockSpec((BLK, 1), lambda r: (r, 0)),
            pl.BlockSpec((1, K), lambda r: (0, 0)),
        ],
        out_specs=[
            pl.BlockSpec((1, 1, BLK), lambda r: (r, 0, 0)),
            pl.BlockSpec(memory_space=pltpu.SMEM, block_shape=(1,),
                         index_map=lambda r: (0,)),
        ],
        out_shape=[
            jax.ShapeDtypeStruct((rows // BLK, 1, BLK), jnp.int32),
            jax.ShapeDtypeStruct((1,), jnp.float32),
        ],
        scratch_shapes=[
            pltpu.VMEM((K, D), jnp.bfloat16),
        ],
    )(xn, ewn, t1, c)


_GW = 128


def _sc_gather(ewn, idx):
    rows = idx.shape[0]
    vector_mesh = plsc.VectorSubcoreMesh(core_axis_name="core",
                                         subcore_axis_name="subcore")
    idx2 = idx.reshape(1, rows)

    @functools.partial(
        pl.kernel,
        out_type=jax.ShapeDtypeStruct((rows, D), jnp.float32),
        mesh=vector_mesh)
    def gather_kernel(ewn_hbm, i_hbm, o_hbm):
        def body(i_vmem, o_vmem):
            pltpu.sync_copy(ewn_hbm.at[i_vmem.at[0]], o_vmem)

        pltpu.emit_pipeline(
            body,
            grid=(rows // _GW,),
            in_specs=[pl.BlockSpec((1, _GW), index_map=lambda i: (0, i))],
            out_specs=[pl.BlockSpec((_GW, D), index_map=lambda i: (i, 0))],
            core_axis_name=("core", "subcore"),
            dimension_semantics=(pltpu.PARALLEL,),
        )(i_hbm, o_hbm)

    return gather_kernel(ewn, idx2)


def kernel(x, emb_w):
    xn = _l2n(x)
    ewn = _l2n(emb_w)
    t1 = jnp.sum(xn ** 2, axis=-1, keepdims=True)
    c = jnp.sum(ewn ** 2, axis=1)[None, :]
    h = N // 2
    idx3a, loss_a = _argmin_call(xn[:h], ewn, t1[:h], c)
    idx_a = idx3a.reshape(h)
    q_a = _sc_gather(ewn, idx_a)
    idx3b, loss_b = _argmin_call(xn[h:], ewn, t1[h:], c)
    idx_b = idx3b.reshape(h)
    q_b = _sc_gather(ewn, idx_b)
    quantized = jnp.concatenate([q_a, q_b], axis=0)
    idx = jnp.concatenate([idx_a, idx_b], axis=0)
    loss = (loss_a + loss_b).reshape(())
    return quantized, loss, idx


# speedup vs baseline: 1.0873x; 1.0873x over previous
import functools

import jax
import jax.numpy as jnp
from jax.experimental import pallas as pl
from jax.experimental.pallas import tpu as pltpu
from jax.experimental.pallas import tpu_sc as plsc

N = 16384
D = 256
K = 8192
BLK = 512
R = N // BLK

_EPS = 1e-12
_LOSS_SCALE = 1.25 / (N * D)


def _l2n(a):
    n = jnp.linalg.norm(a, axis=-1, keepdims=True)
    return a / jnp.maximum(n, _EPS)


def _argmin_body(xn_ref, emb_ref, t1_ref, c_ref, idx_ref, loss_ref, ebf_ref):
    r = pl.program_id(0)

    @pl.when(r == 0)
    def _():
        ebf_ref[...] = emb_ref[...].astype(jnp.bfloat16)
        loss_ref[0] = 0.0

    xn = xn_ref[...]
    dots = jax.lax.dot_general(
        xn.astype(jnp.bfloat16), ebf_ref[...],
        (((1,), (1,)), ((), ())),
        preferred_element_type=jnp.float32)
    d = t1_ref[...] + c_ref[...] - 2.0 * dots

    b1 = K // 2
    m0 = jnp.min(d[:, :b1], axis=1)
    i0 = jnp.argmin(d[:, :b1], axis=1).astype(jnp.int32)
    m1 = jnp.min(d[:, b1:], axis=1)
    i1 = jnp.argmin(d[:, b1:], axis=1).astype(jnp.int32) + b1

    def _round_bf16(v):
        u = jax.lax.bitcast_convert_type(v, jnp.uint32)
        lsb = jax.lax.shift_right_logical(u, jnp.uint32(16)) & jnp.uint32(1)
        rr = (u + jnp.uint32(0x7FFF) + lsb) & jnp.uint32(0xFFFF0000)
        return jax.lax.bitcast_convert_type(rr, jnp.float32)

    upd = m1 < _round_bf16(m0)
    idx_ref[0, 0, :] = jnp.where(upd, i1, i0)
    loss_ref[0] += jnp.sum(jnp.where(upd, m1, m0))

    @pl.when(r == R - 1)
    def _():
        loss_ref[0] = loss_ref[0] * _LOSS_SCALE


def _argmin_call(xn, ewn, t1, c):
    return pl.pallas_call(
        _argmin_body,
        grid=(R,),
        in_specs=[
            pl.BlockSpec((BLK, D), lambda r: (r, 0)),
            pl.BlockSpec((K, D), lambda r: (0, 0)),
            pl.BlockSpec((BLK, 1), lambda r: (r, 0)),
            pl.BlockSpec((1, K), lambda r: (0, 0)),
        ],
        out_specs=[
            pl.BlockSpec((1, 1, BLK), lambda r: (r, 0, 0)),
            pl.BlockSpec(memory_space=pltpu.SMEM, block_shape=(1,),
                         index_map=lambda r: (0,)),
        ],
        out_shape=[
            jax.ShapeDtypeStruct((R, 1, BLK), jnp.int32),
            jax.ShapeDtypeStruct((1,), jnp.float32),
        ],
        scratch_shapes=[
            pltpu.VMEM((K, D), jnp.bfloat16),
        ],
    )(xn, ewn, t1, c)


_GW = 128


def _sc_gather(ewn, idx):
    vector_mesh = plsc.VectorSubcoreMesh(core_axis_name="core",
                                         subcore_axis_name="subcore")
    idx2 = idx.reshape(1, N)

    @functools.partial(
        pl.kernel,
        out_type=jax.ShapeDtypeStruct((N, D), jnp.float32),
        mesh=vector_mesh)
    def gather_kernel(ewn_hbm, i_hbm, o_hbm):
        def body(i_vmem, o_vmem):
            pltpu.sync_copy(ewn_hbm.at[i_vmem.at[0]], o_vmem)

        pltpu.emit_pipeline(
            body,
            grid=(N // _GW,),
            in_specs=[pl.BlockSpec((1, _GW), index_map=lambda i: (0, i))],
            out_specs=[pl.BlockSpec((_GW, D), index_map=lambda i: (i, 0))],
            core_axis_name=("core", "subcore"),
            dimension_semantics=(pltpu.PARALLEL,),
        )(i_hbm, o_hbm)

    return gather_kernel(ewn, idx2)


def kernel(x, emb_w):
    xn = _l2n(x)
    ewn = _l2n(emb_w)
    t1 = jnp.sum(xn ** 2, axis=-1, keepdims=True)
    c = jnp.sum(ewn ** 2, axis=1)[None, :]
    idx3, loss = _argmin_call(xn, ewn, t1, c)
    idx = idx3.reshape(N)
    quantized = _sc_gather(ewn, idx)
    return quantized, loss.reshape(()), idx
